# baseline (speedup 1.0000x reference)
import jax
import jax.numpy as jnp
from jax import lax
from jax.experimental import pallas as pl
from jax.experimental.pallas import tpu as pltpu
from jax.experimental.pallas import tpu_sc as plsc

B = 4
H = W = 192
C = 96
HW = H * W
P = (2 * H) * (2 * W)
CB = 4
NBLK = C // CB
NC = 2
NS = 16
LANES = 16
POS_PER_TILE = HW // NS
NELEM = POS_PER_TILE * CB
ROWS_PER_TILE = P // NS
ACC_WORDS = CB * P
ZCHUNK = NELEM
NTASK = B * NBLK
VPC = POS_PER_TILE // LANES


def _body(mask_hbm, upd_hbm, out_hbm, mbuf, midx, ubuf, zbuf, acc):
  cid = lax.axis_index("c")
  sid = lax.axis_index("s")

  zeros16 = jnp.zeros((LANES,), jnp.float32)

  def zfill(i, _):
    zbuf[pl.ds(i * LANES, LANES)] = zeros16
    return 0
  lax.fori_loop(0, ZCHUNK // LANES, zfill, 0)

  def zero_acc():
    base = sid * (ACC_WORDS // NS)
    for j in range(ACC_WORDS // NS // ZCHUNK):
      pltpu.sync_copy(zbuf, acc.at[pl.ds(base + j * ZCHUNK, ZCHUNK)])

  zero_acc()
  plsc.subcore_barrier()

  def task(t, _):
    task_id = t * NC + cid
    b = task_id // NBLK
    blk = task_id % NBLK
    pos0 = sid * POS_PER_TILE
    c0 = blk * CB

    for k in range(CB):
      pltpu.sync_copy(
          mask_hbm.at[b, c0 + k, pl.ds(pos0, POS_PER_TILE)],
          mbuf.at[pl.ds(k * POS_PER_TILE, POS_PER_TILE)],
      )
      pltpu.sync_copy(
          upd_hbm.at[b, c0 + k, pl.ds(pos0, POS_PER_TILE)],
          ubuf.at[pl.ds(k * POS_PER_TILE, POS_PER_TILE)],
      )

    cvec = jnp.full((LANES,), C, jnp.int32)
    for k in range(CB):
      kvec = jnp.full((LANES,), k * P, jnp.int32)

      def compute(i, _, kvec=kvec, k=k):
        j = (k * VPC + i) * LANES
        m = mbuf[pl.ds(j, LANES)]
        midx[pl.ds(j, LANES)] = lax.div(m, cvec) + kvec
        return 0
      lax.fori_loop(0, VPC, compute, 0)

    for j in range(NS):
      @pl.when(sid == j)
      def _():
        pltpu.sync_copy(ubuf, acc.at[midx], add=True)
      plsc.subcore_barrier()
    plsc.subcore_barrier()

    r0 = sid * ROWS_PER_TILE
    for k in range(CB):
      pltpu.sync_copy(
          acc.at[pl.ds(k * P + r0, ROWS_PER_TILE)],
          out_hbm.at[b, c0 + k, pl.ds(r0, ROWS_PER_TILE)],
      )
    zero_acc()
    plsc.subcore_barrier()
    return 0

  lax.fori_loop(0, NTASK // NC, task, 0)


@jax.jit
def kernel(updates, mask):
  mask_t = jnp.transpose(mask.astype(jnp.int32).reshape(B, HW, C), (0, 2, 1))
  upd_t = jnp.transpose(updates.reshape(B, HW, C), (0, 2, 1))
  mesh = plsc.VectorSubcoreMesh(
      core_axis_name="c", subcore_axis_name="s", num_cores=NC, num_subcores=NS
  )
  out_t = pl.kernel(
      _body,
      out_type=jax.ShapeDtypeStruct((B, C, P), jnp.float32),
      mesh=mesh,
      scratch_types=[
          pltpu.VMEM((NELEM,), jnp.int32),
          pltpu.VMEM((NELEM,), jnp.int32),
          pltpu.VMEM((NELEM,), jnp.float32),
          pltpu.VMEM((ZCHUNK,), jnp.float32),
          pltpu.VMEM_SHARED((ACC_WORDS,), jnp.float32),
      ],
  )(mask_t, upd_t)
  return jnp.transpose(out_t, (0, 2, 1)).reshape(B, 2 * H, 2 * W, C)


# speedup vs baseline: 2.4059x; 2.4059x over previous
import jax
import jax.numpy as jnp
from jax import lax
from jax.experimental import pallas as pl
from jax.experimental.pallas import tpu as pltpu
from jax.experimental.pallas import tpu_sc as plsc

B = 4
H = W = 192
C = 96
HW = H * W
P = (2 * H) * (2 * W)
CB = 4
NBLK = C // CB
NC = 2
NS = 16
LANES = 16
POS_PER_TILE = HW // NS
NELEM = POS_PER_TILE * CB
ROWS_PER_TILE = P // NS
ACC_WORDS = CB * P
ZCHUNK = NELEM
NTASK = B * NBLK
VPC = POS_PER_TILE // LANES


def _body(mask_hbm, upd_hbm, out_hbm, mbuf, midx, ubuf, zbuf, acc):
  cid = lax.axis_index("c")
  sid = lax.axis_index("s")

  zeros16 = jnp.zeros((LANES,), jnp.float32)

  def zfill(i, _):
    zbuf[pl.ds(i * LANES, LANES)] = zeros16
    return 0
  lax.fori_loop(0, ZCHUNK // LANES, zfill, 0)

  def zero_acc():
    base = sid * (ACC_WORDS // NS)
    for j in range(ACC_WORDS // NS // ZCHUNK):
      pltpu.sync_copy(zbuf, acc.at[pl.ds(base + j * ZCHUNK, ZCHUNK)])

  zero_acc()
  plsc.subcore_barrier()

  kch = lax.bitwise_and(sid, CB - 1)
  q = lax.shift_right_logical(sid, 2)
  NQ = NS // CB
  POSQ = HW // NQ

  def task(t, _):
    task_id = t * NC + cid
    b = task_id // NBLK
    blk = task_id % NBLK
    pos0 = q * POSQ
    c0 = blk * CB

    pltpu.sync_copy(mask_hbm.at[b, c0 + kch, pl.ds(pos0, POSQ)], mbuf)
    pltpu.sync_copy(upd_hbm.at[b, c0 + kch, pl.ds(pos0, POSQ)], ubuf)

    cvec = jnp.full((LANES,), C, jnp.int32)
    for k in range(CB):
      @pl.when(kch == k)
      def _(k=k):
        kvec = jnp.full((LANES,), k * P, jnp.int32)

        def compute(i, _):
          m = mbuf[pl.ds(i * LANES, LANES)]
          midx[pl.ds(i * LANES, LANES)] = lax.div(m, cvec) + kvec
          return 0
        lax.fori_loop(0, NELEM // LANES, compute, 0)

    pltpu.sync_copy(ubuf, acc.at[midx], add=True)
    plsc.subcore_barrier()

    r0 = sid * ROWS_PER_TILE
    for k in range(CB):
      pltpu.sync_copy(
          acc.at[pl.ds(k * P + r0, ROWS_PER_TILE)],
          out_hbm.at[b, c0 + k, pl.ds(r0, ROWS_PER_TILE)],
      )
    plsc.subcore_barrier()
    zero_acc()
    plsc.subcore_barrier()
    return 0

  lax.fori_loop(0, NTASK // NC, task, 0)


@jax.jit
def kernel(updates, mask):
  mask_t = jnp.transpose(mask.astype(jnp.int32).reshape(B, HW, C), (0, 2, 1))
  upd_t = jnp.transpose(updates.reshape(B, HW, C), (0, 2, 1))
  mesh = plsc.VectorSubcoreMesh(
      core_axis_name="c", subcore_axis_name="s", num_cores=NC, num_subcores=NS
  )
  out_t = pl.kernel(
      _body,
      out_type=jax.ShapeDtypeStruct((B, C, P), jnp.float32),
      mesh=mesh,
      scratch_types=[
          pltpu.VMEM((NELEM,), jnp.int32),
          pltpu.VMEM((NELEM,), jnp.int32),
          pltpu.VMEM((NELEM,), jnp.float32),
          pltpu.VMEM((ZCHUNK,), jnp.float32),
          pltpu.VMEM_SHARED((ACC_WORDS,), jnp.float32),
      ],
  )(mask_t, upd_t)
  return jnp.transpose(out_t, (0, 2, 1)).reshape(B, 2 * H, 2 * W, C)


# speedup vs baseline: 2.6228x; 1.0901x over previous
import jax
import jax.numpy as jnp
from jax import lax
from jax.experimental import pallas as pl
from jax.experimental.pallas import tpu as pltpu
from jax.experimental.pallas import tpu_sc as plsc

B = 4
H = W = 192
C = 96
HW = H * W
P = (2 * H) * (2 * W)
CB = 4
NBLK = C // CB
NC = 2
NS = 16
LANES = 16
NQ = NS // CB
POSQ = HW // NQ
NELEM = POSQ
ACC_WORDS = CB * P
SLICE = ACC_WORDS // NS
NTASK = B * NBLK
TPC = NTASK // NC


def _body(mask_hbm, upd_hbm, out_hbm,
          mbuf0, mbuf1, ubuf0, ubuf1, midx, zbuf, acc, msem, usem):
  cid = lax.axis_index("c")
  sid = lax.axis_index("s")
  kch = lax.bitwise_and(sid, CB - 1)
  q = lax.shift_right_logical(sid, 2)
  kd = lax.shift_right_logical(sid, 2)
  rd = lax.bitwise_and(sid, CB - 1) * SLICE

  zeros16 = jnp.zeros((LANES,), jnp.float32)

  def zfill(i, _):
    zbuf[pl.ds(i * LANES, LANES)] = zeros16
    return 0
  lax.fori_loop(0, SLICE // LANES, zfill, 0)

  def zero_acc():
    pltpu.sync_copy(zbuf, acc.at[pl.ds(sid * SLICE, SLICE)])

  def src_slices(t):
    task_id = t * NC + cid
    b = task_id // NBLK
    c = (task_id % NBLK) * CB + kch
    pos0 = q * POSQ
    return b, c, pos0

  def start_in(t, mb, ub):
    b, c, pos0 = src_slices(t)
    pltpu.async_copy(mask_hbm.at[b, c, pl.ds(pos0, POSQ)], mb, msem)
    pltpu.async_copy(upd_hbm.at[b, c, pl.ds(pos0, POSQ)], ub, usem)

  def wait_in(t, mb, ub):
    b, c, pos0 = src_slices(t)
    pltpu.make_async_copy(
        mask_hbm.at[b, c, pl.ds(pos0, POSQ)], mb, msem
    ).wait()
    pltpu.make_async_copy(
        upd_hbm.at[b, c, pl.ds(pos0, POSQ)], ub, usem
    ).wait()

  zero_acc()
  start_in(0, mbuf0, ubuf0)

  cvec = jnp.full((LANES,), C, jnp.int32)

  def compute_idx(mb):
    for k in range(CB):
      @pl.when(kch == k)
      def _(k=k):
        kvec = jnp.full((LANES,), k * P, jnp.int32)

        def compute(i, _):
          m = mb[pl.ds(i * LANES, LANES)]
          midx[pl.ds(i * LANES, LANES)] = lax.div(m, cvec) + kvec
          return 0
        lax.fori_loop(0, NELEM // LANES, compute, 0)

  def task(t, _):
    tb = lax.bitwise_and(t, 1)
    even = tb == 0

    @pl.when(even)
    def _():
      wait_in(t, mbuf0, ubuf0)
      compute_idx(mbuf0)

    @pl.when(~even)
    def _():
      wait_in(t, mbuf1, ubuf1)
      compute_idx(mbuf1)

    plsc.subcore_barrier()

    @pl.when(even)
    def _():
      pltpu.sync_copy(ubuf0, acc.at[midx], add=True)

    @pl.when(~even)
    def _():
      pltpu.sync_copy(ubuf1, acc.at[midx], add=True)

    plsc.subcore_barrier()

    tn = lax.min(t + 1, TPC - 1)

    @pl.when(even)
    def _():
      start_in(tn, mbuf1, ubuf1)

    @pl.when(~even)
    def _():
      start_in(tn, mbuf0, ubuf0)

    task_id = t * NC + cid
    b = task_id // NBLK
    c0 = (task_id % NBLK) * CB
    pltpu.sync_copy(
        acc.at[pl.ds(sid * SLICE, SLICE)],
        out_hbm.at[b, c0 + kd, pl.ds(rd, SLICE)],
    )
    zero_acc()
    return 0

  lax.fori_loop(0, TPC, task, 0)
  wait_in(TPC - 1, mbuf0, ubuf0)


@jax.jit
def kernel(updates, mask):
  mask_t = jnp.transpose(mask.astype(jnp.int32).reshape(B, HW, C), (0, 2, 1))
  upd_t = jnp.transpose(updates.reshape(B, HW, C), (0, 2, 1))
  mesh = plsc.VectorSubcoreMesh(
      core_axis_name="c", subcore_axis_name="s", num_cores=NC, num_subcores=NS
  )
  out_t = pl.kernel(
      _body,
      out_type=jax.ShapeDtypeStruct((B, C, P), jnp.float32),
      mesh=mesh,
      scratch_types=[
          pltpu.VMEM((NELEM,), jnp.int32),
          pltpu.VMEM((NELEM,), jnp.int32),
          pltpu.VMEM((NELEM,), jnp.float32),
          pltpu.VMEM((NELEM,), jnp.float32),
          pltpu.VMEM((NELEM,), jnp.int32),
          pltpu.VMEM((SLICE,), jnp.float32),
          pltpu.VMEM_SHARED((ACC_WORDS,), jnp.float32),
          pltpu.SemaphoreType.DMA,
          pltpu.SemaphoreType.DMA,
      ],
  )(mask_t, upd_t)
  return jnp.transpose(out_t, (0, 2, 1)).reshape(B, 2 * H, 2 * W, C)


# speedup vs baseline: 2.6893x; 1.0254x over previous
import jax
import jax.numpy as jnp
from jax import lax
from jax.experimental import pallas as pl
from jax.experimental.pallas import tpu as pltpu
from jax.experimental.pallas import tpu_sc as plsc

B = 4
H = W = 192
C = 96
HW = H * W
P = (2 * H) * (2 * W)
CB = 4
NBLK = C // CB
NC = 2
NS = 16
LANES = 16
NQ = NS // CB
POSQ = HW // NQ
NELEM = POSQ
ACC_WORDS = CB * P
SLICE = ACC_WORDS // NS
NTASK = B * NBLK
TPC = NTASK // NC


def _body(mask_hbm, upd_hbm, out_hbm,
          mbuf0, mbuf1, ubuf0, ubuf1, midx, zbuf, acc, msem, usem, dsem):
  cid = lax.axis_index("c")
  sid = lax.axis_index("s")
  kch = lax.bitwise_and(sid, CB - 1)
  q = lax.shift_right_logical(sid, 2)
  kd = lax.shift_right_logical(sid, 2)
  rd = lax.bitwise_and(sid, CB - 1) * SLICE

  zeros16 = jnp.zeros((LANES,), jnp.float32)

  def zfill(i, _):
    zbuf[pl.ds(i * LANES, LANES)] = zeros16
    return 0
  lax.fori_loop(0, SLICE // LANES, zfill, 0)

  def zero_acc():
    pltpu.sync_copy(zbuf, acc.at[pl.ds(sid * SLICE, SLICE)])

  def src_slices(t):
    task_id = t * NC + cid
    b = task_id // NBLK
    c = (task_id % NBLK) * CB + kch
    pos0 = q * POSQ
    return b, c, pos0

  def start_in(t, mb, ub):
    b, c, pos0 = src_slices(t)
    pltpu.async_copy(mask_hbm.at[b, c, pl.ds(pos0, POSQ)], mb, msem)
    pltpu.async_copy(upd_hbm.at[b, c, pl.ds(pos0, POSQ)], ub, usem)

  def wait_in(t, mb, ub):
    b, c, pos0 = src_slices(t)
    pltpu.make_async_copy(
        mask_hbm.at[b, c, pl.ds(pos0, POSQ)], mb, msem
    ).wait()
    pltpu.make_async_copy(
        upd_hbm.at[b, c, pl.ds(pos0, POSQ)], ub, usem
    ).wait()

  cvec = jnp.full((LANES,), C, jnp.int32)

  def compute_idx(mb):
    for k in range(CB):
      @pl.when(kch == k)
      def _(k=k):
        kvec = jnp.full((LANES,), k * P, jnp.int32)

        def compute(i, _):
          m = mb[pl.ds(i * LANES, LANES)]
          midx[pl.ds(i * LANES, LANES)] = lax.div(m, cvec) + kvec
          return 0
        lax.fori_loop(0, NELEM // LANES, compute, 0)

  def drain_ref(t):
    task_id = t * NC + cid
    b = task_id // NBLK
    c0 = (task_id % NBLK) * CB
    return out_hbm.at[b, c0 + kd, pl.ds(rd, SLICE)]

  zero_acc()
  start_in(0, mbuf0, ubuf0)
  wait_in(0, mbuf0, ubuf0)
  compute_idx(mbuf0)

  def task(t, _):
    even = lax.bitwise_and(t, 1) == 0

    plsc.subcore_barrier()

    @pl.when(even)
    def _():
      pltpu.sync_copy(ubuf0, acc.at[midx], add=True)

    @pl.when(~even)
    def _():
      pltpu.sync_copy(ubuf1, acc.at[midx], add=True)

    plsc.subcore_barrier()

    tn = lax.min(t + 1, TPC - 1)

    @pl.when(even)
    def _():
      start_in(tn, mbuf1, ubuf1)

    @pl.when(~even)
    def _():
      start_in(tn, mbuf0, ubuf0)

    pltpu.async_copy(acc.at[pl.ds(sid * SLICE, SLICE)], drain_ref(t), dsem)

    @pl.when(even)
    def _():
      wait_in(tn, mbuf1, ubuf1)
      compute_idx(mbuf1)

    @pl.when(~even)
    def _():
      wait_in(tn, mbuf0, ubuf0)
      compute_idx(mbuf0)

    pltpu.make_async_copy(
        acc.at[pl.ds(sid * SLICE, SLICE)], drain_ref(t), dsem
    ).wait()
    zero_acc()
    return 0

  lax.fori_loop(0, TPC, task, 0)


@jax.jit
def kernel(updates, mask):
  mask_t = jnp.transpose(mask.astype(jnp.int32).reshape(B, HW, C), (0, 2, 1))
  upd_t = jnp.transpose(updates.reshape(B, HW, C), (0, 2, 1))
  mesh = plsc.VectorSubcoreMesh(
      core_axis_name="c", subcore_axis_name="s", num_cores=NC, num_subcores=NS
  )
  out_t = pl.kernel(
      _body,
      out_type=jax.ShapeDtypeStruct((B, C, P), jnp.float32),
      mesh=mesh,
      scratch_types=[
          pltpu.VMEM((NELEM,), jnp.int32),
          pltpu.VMEM((NELEM,), jnp.int32),
          pltpu.VMEM((NELEM,), jnp.float32),
          pltpu.VMEM((NELEM,), jnp.float32),
          pltpu.VMEM((NELEM,), jnp.int32),
          pltpu.VMEM((SLICE,), jnp.float32),
          pltpu.VMEM_SHARED((ACC_WORDS,), jnp.float32),
          pltpu.SemaphoreType.DMA,
          pltpu.SemaphoreType.DMA,
          pltpu.SemaphoreType.DMA,
      ],
  )(mask_t, upd_t)
  return jnp.transpose(out_t, (0, 2, 1)).reshape(B, 2 * H, 2 * W, C)


# speedup vs baseline: 3.1313x; 1.1644x over previous
import jax
import jax.numpy as jnp
from jax import lax
from jax.experimental import pallas as pl
from jax.experimental.pallas import tpu as pltpu
from jax.experimental.pallas import tpu_sc as plsc

B = 4
H = W = 192
C = 96
HW = H * W
P = (2 * H) * (2 * W)
CB = 2
NBLK = C // CB
NC = 2
NS = 16
LANES = 16
NQ = NS // CB
POSQ = HW // NQ
NELEM = POSQ
SLAB = CB * P
SLICE = SLAB // NS
NTASK = B * NBLK
TPC = NTASK // NC


def _body(mask_hbm, upd_hbm, out_hbm,
          mbuf0, mbuf1, ubuf0, ubuf1, midx0, midx1, zbuf, accA, accB,
          msem, usem, ssem, dsemA, dsemB, zsemA, zsemB):
  cid = lax.axis_index("c")
  sid = lax.axis_index("s")
  kch = lax.bitwise_and(sid, CB - 1)
  q = lax.shift_right_logical(sid, 1)
  kd = lax.shift_right_logical(sid, 3)
  rd = lax.bitwise_and(sid, NQ - 1) * SLICE

  zeros16 = jnp.zeros((LANES,), jnp.float32)

  def zfill(i, _):
    zbuf[pl.ds(i * LANES, LANES)] = zeros16
    return 0
  lax.fori_loop(0, SLICE // LANES, zfill, 0)

  def zero_start(acc, zsem):
    pltpu.async_copy(zbuf, acc.at[pl.ds(sid * SLICE, SLICE)], zsem)

  def zero_wait(acc, zsem):
    pltpu.make_async_copy(
        zbuf, acc.at[pl.ds(sid * SLICE, SLICE)], zsem
    ).wait()

  def src_slices(t):
    task_id = t * NC + cid
    b = task_id // NBLK
    c = (task_id % NBLK) * CB + kch
    pos0 = q * POSQ
    return b, c, pos0

  def start_in(t, mb, ub):
    b, c, pos0 = src_slices(t)
    pltpu.async_copy(mask_hbm.at[b, c, pl.ds(pos0, POSQ)], mb, msem)
    pltpu.async_copy(upd_hbm.at[b, c, pl.ds(pos0, POSQ)], ub, usem)

  def wait_in(t, mb, ub):
    b, c, pos0 = src_slices(t)
    pltpu.make_async_copy(
        mask_hbm.at[b, c, pl.ds(pos0, POSQ)], mb, msem
    ).wait()
    pltpu.make_async_copy(
        upd_hbm.at[b, c, pl.ds(pos0, POSQ)], ub, usem
    ).wait()

  cvec = jnp.full((LANES,), C, jnp.int32)

  def compute_idx(mb, mx):
    for k in range(CB):
      @pl.when(kch == k)
      def _(k=k):
        kvec = jnp.full((LANES,), k * P, jnp.int32)

        def compute(i, _):
          m = mb[pl.ds(i * LANES, LANES)]
          mx[pl.ds(i * LANES, LANES)] = lax.div(m, cvec) + kvec
          return 0
        lax.fori_loop(0, NELEM // LANES, compute, 0)

  def drain_ref(t):
    task_id = t * NC + cid
    b = task_id // NBLK
    c0 = (task_id % NBLK) * CB
    return out_hbm.at[b, c0 + kd, pl.ds(rd, SLICE)]

  def drain_start(t, acc, dsem):
    pltpu.async_copy(acc.at[pl.ds(sid * SLICE, SLICE)], drain_ref(t), dsem)

  def drain_wait(t, acc, dsem):
    pltpu.make_async_copy(
        acc.at[pl.ds(sid * SLICE, SLICE)], drain_ref(t), dsem
    ).wait()

  zero_start(accA, zsemA)
  zero_start(accB, zsemB)
  start_in(0, mbuf0, ubuf0)
  wait_in(0, mbuf0, ubuf0)
  compute_idx(mbuf0, midx0)
  start_in(1, mbuf1, ubuf1)

  def step(t, cur, nxt):
    mb_c, ub_c, mx_c, acc_c, dsem_c, zsem_c = cur
    mb_n, ub_n, mx_n, acc_n, dsem_n, zsem_n = nxt

    zero_wait(acc_c, zsem_c)
    plsc.subcore_barrier()

    pltpu.async_copy(ub_c, acc_c.at[mx_c], ssem, add=True)
    tn = lax.min(t + 1, TPC - 1)
    wait_in(tn, mb_n, ub_n)
    compute_idx(mb_n, mx_n)
    pltpu.make_async_copy(ub_c, acc_c.at[mx_c], ssem).wait()
    plsc.subcore_barrier()

    drain_start(t, acc_c, dsem_c)

    @pl.when(t > 0)
    def _():
      drain_wait(t - 1, acc_n, dsem_n)
      zero_start(acc_n, zsem_n)

    start_in(lax.min(t + 2, TPC - 1), mb_c, ub_c)
    return 0

  bufs0 = (mbuf0, ubuf0, midx0, accA, dsemA, zsemA)
  bufs1 = (mbuf1, ubuf1, midx1, accB, dsemB, zsemB)

  def task(t, _):
    even = lax.bitwise_and(t, 1) == 0

    @pl.when(even)
    def _():
      step(t, bufs0, bufs1)

    @pl.when(~even)
    def _():
      step(t, bufs1, bufs0)

    return 0

  lax.fori_loop(0, TPC, task, 0)

  drain_wait(TPC - 1, accB, dsemB)
  zero_wait(accA, zsemA)
  wait_in(TPC - 1, mbuf1, ubuf1)


@jax.jit
def kernel(updates, mask):
  mask_t = jnp.transpose(mask.astype(jnp.int32).reshape(B, HW, C), (0, 2, 1))
  upd_t = jnp.transpose(updates.reshape(B, HW, C), (0, 2, 1))
  mesh = plsc.VectorSubcoreMesh(
      core_axis_name="c", subcore_axis_name="s", num_cores=NC, num_subcores=NS
  )
  out_t = pl.kernel(
      _body,
      out_type=jax.ShapeDtypeStruct((B, C, P), jnp.float32),
      mesh=mesh,
      scratch_types=[
          pltpu.VMEM((NELEM,), jnp.int32),
          pltpu.VMEM((NELEM,), jnp.int32),
          pltpu.VMEM((NELEM,), jnp.float32),
          pltpu.VMEM((NELEM,), jnp.float32),
          pltpu.VMEM((NELEM,), jnp.int32),
          pltpu.VMEM((NELEM,), jnp.int32),
          pltpu.VMEM((SLICE,), jnp.float32),
          pltpu.VMEM_SHARED((SLAB,), jnp.float32),
          pltpu.VMEM_SHARED((SLAB,), jnp.float32),
          pltpu.SemaphoreType.DMA,
          pltpu.SemaphoreType.DMA,
          pltpu.SemaphoreType.DMA,
          pltpu.SemaphoreType.DMA,
          pltpu.SemaphoreType.DMA,
          pltpu.SemaphoreType.DMA,
          pltpu.SemaphoreType.DMA,
      ],
  )(mask_t, upd_t)
  return jnp.transpose(out_t, (0, 2, 1)).reshape(B, 2 * H, 2 * W, C)
